# baseline (speedup 1.0000x reference)
import functools

import numpy as np

import jax
import jax.numpy as jnp
from jax import lax
from jax.experimental import pallas as pl
from jax.experimental.pallas import tpu as pltpu
from jax.experimental.pallas import tpu_sc as plsc

_B = 16384
_NC = 2
_NS = 16
_NW = _NC * _NS
_RPW = _B // _NW
_CH = 128
_NCH = _RPW // _CH
_PIECE = 32

_OUT_D = 322
_STORE_OFF = tuple(16 * g for g in range(20)) + (306,)
_NG = len(_STORE_OFF)

_R2, _R3, _R5, _R6 = 0, 1218, 1906, 2238
_VB = 3069


def _build_map() -> np.ndarray:
    spans = (
        (0, 7, 'T01'), (7, 57, 'B'), (57, 107, 'B'), (107, 109, 'W4'),
        (109, 159, 'B'), (159, 209, 'B'), (209, 259, 'B'),
        (259, 309, 'B'), (309, 322, 'XN'),
    )

    def src(c):
        for lo, hi, s in spans:
            if lo <= c < hi:
                return s, c - lo
        raise AssertionError(c)

    vecs = np.zeros((45, 16), np.int64)
    for g in range(_NG):
        for l in range(16):
            kind, ic = src(_STORE_OFF[g] + l)
            if kind == 'B':
                vecs[2 * g, l] = ic >> 1
                vecs[2 * g + 1, l] = 16 if ic % 2 == 0 else 0
    for l in range(7):
        vecs[42, l] = l
    vecs[43, 11], vecs[43, 12] = 0, 1
    for l in range(3, 16):
        vecs[44, l] = l - 3
    tab = np.zeros((6, 128), np.int32)
    for v in range(45):
        tab[v // 8, (v % 8) * 16:(v % 8) * 16 + 16] = vecs[v]
    return tab


_MAP = _build_map()
_HI_MASK = np.int32(np.uint32(0xFFFF0000).view(np.int32))


def _body(xn16, idx, vmap, Tblob, T01, W4p, out,
          idx_v, map_v, tbv, t01v, w4v, xnb0, xnb1,
          asm0, asm1, gsem0, gsem1, osem):
    wid = lax.axis_index("s") * _NC + lax.axis_index("c")
    base = wid * _RPW
    xnbs = (xnb0, xnb1)
    gsems = (gsem0, gsem1)
    asms = (asm0, asm1)

    pltpu.sync_copy(idx.at[wid], idx_v)
    pltpu.sync_copy(vmap, map_v)
    pltpu.sync_copy(Tblob, tbv)
    pltpu.sync_copy(T01, t01v)
    pltpu.sync_copy(W4p, w4v)

    fb = [map_v[v // 8, pl.ds((v % 8) * 16, 16)] for v in range(45)]
    lane = jax.lax.iota(jnp.int32, 16)
    m01 = lane < 7
    mw4 = (lane >= 11) & (lane < 13)
    mxn = lane < 3
    m1 = lane < 1
    m3 = lane < 3
    m9 = lane < 9
    m13 = lane >= 13
    m15 = lane < 15
    himask = jnp.full((16,), _HI_MASK, jnp.int32)

    def issue_xn(k):
        rows = pl.ds(base + k * _CH, _CH)
        return [pltpu.async_copy(xn16.at[rows, :], xnbs[k % 2],
                                 gsems[k % 2])]

    def repack_piece(k, piece):
        xnb = xnbs[k % 2]
        asm = asms[piece % 2]

        @pl.loop(piece * _PIECE, (piece + 1) * _PIECE)
        def _(r):
            a = r - piece * _PIECE
            rvec = jnp.full((16,), r, jnp.int32)

            def bidx(fieldrow, off):
                iv = plsc.load_gather(
                    idx_v, [jnp.full((16,), fieldrow, jnp.int32), rvec])
                return iv + off if off else iv

            i01 = bidx(0 * _NCH + k, 0)
            i2 = bidx(1 * _NCH + k, _R2)
            i3 = bidx(2 * _NCH + k, _R3)
            i4 = bidx(3 * _NCH + k, 0)
            i5 = bidx(4 * _NCH + k, _R5)
            i6 = bidx(5 * _NCH + k, _R6)

            rowsel = {
                0: i2, 1: i2, 2: i2, 3: jnp.where(m9, i2, i3),
                4: i3, 5: i3, 6: jnp.where(m13, i5, i3),
                7: i5, 8: i5, 9: jnp.where(m15, i5, i6),
                10: i6, 11: i6, 12: i6, 13: jnp.where(m1, i6, i5),
                14: i5, 15: i5, 16: jnp.where(m3, i5, i6),
                17: i6, 18: i6, 19: i6, 20: i6,
            }
            for g in range(_NG):
                w = plsc.load_gather(tbv, [rowsel[g], fb[2 * g]])
                v = plsc.bitcast((w << fb[2 * g + 1]) & himask, jnp.float32)
                if g == 0:
                    v = jnp.where(
                        m01, plsc.load_gather(t01v, [i01, fb[42]]), v)
                elif g == 6:
                    v = jnp.where(
                        mw4, plsc.load_gather(w4v, [i4, fb[43]]), v)
                elif g == 20:
                    v = jnp.where(
                        mxn, v, plsc.load_gather(xnb, [rvec, fb[44]]))
                asm[a, pl.ds(_STORE_OFF[g], 16)] = v

    pend = issue_xn(0)
    wb = {}
    piece_id = 0
    for k in range(_NCH):
        nxt = issue_xn(k + 1) if k + 1 < _NCH else []
        for c in pend:
            c.wait()
        pend = nxt
        for piece in range(_CH // _PIECE):
            if piece_id % 2 in wb:
                wb.pop(piece_id % 2).wait()
            repack_piece(k, piece)
            wb[piece_id % 2] = pltpu.async_copy(
                asms[piece_id % 2],
                out.at[pl.ds(base + k * _CH + piece * _PIECE, _PIECE), :],
                osem)
            piece_id += 1
    for c in wb.values():
        c.wait()


_sc_embed = functools.partial(
    pl.kernel,
    out_type=jax.ShapeDtypeStruct((_B, _OUT_D), jnp.float32),
    mesh=plsc.VectorSubcoreMesh(core_axis_name="c", subcore_axis_name="s"),
    compiler_params=pltpu.CompilerParams(use_tc_tiling_on_sc=False,
                                         needs_layout_passes=False),
    scratch_types=[
        pltpu.VMEM((6 * _NCH, _CH), jnp.int32),
        pltpu.VMEM((6, 128), jnp.int32),
        pltpu.VMEM((_VB, 32), jnp.int32),
        pltpu.VMEM((40, 16), jnp.float32),
        pltpu.VMEM((4, 16), jnp.float32),
        pltpu.VMEM((_CH, 16), jnp.float32),
        pltpu.VMEM((_CH, 16), jnp.float32),
        pltpu.VMEM((_PIECE, _OUT_D), jnp.float32),
        pltpu.VMEM((_PIECE, _OUT_D), jnp.float32),
        pltpu.SemaphoreType.DMA,
        pltpu.SemaphoreType.DMA,
        pltpu.SemaphoreType.DMA,
    ],
)(_body)


def kernel(x_num, x_cat, W0, W1, W2, W3, W4, W5, W6):
    f32 = jnp.float32
    Tb = jnp.concatenate([W2, W3, W5, W6], axis=0).astype(jnp.bfloat16)
    Tb32 = jax.lax.bitcast_convert_type(Tb.reshape(_VB, 25, 2), jnp.int32)
    Tblob = jnp.concatenate([Tb32, jnp.zeros((_VB, 7), jnp.int32)], axis=1)
    T01 = jnp.concatenate([
        jnp.repeat(W0.astype(f32), 8, axis=0),
        jnp.tile(W1.astype(f32), (5, 1)),
        jnp.zeros((40, 9), f32),
    ], axis=1)
    W4p = jnp.concatenate([W4.astype(f32), jnp.zeros((4, 14), f32)], axis=1)
    xn16 = jnp.concatenate([x_num.astype(f32), jnp.zeros((_B, 3), f32)], axis=1)

    xc = x_cat.astype(jnp.int32)
    cols = [xc[:, 0] * 8 + xc[:, 1], xc[:, 2], xc[:, 3], xc[:, 4], xc[:, 5],
            xc[:, 6]]
    xi = jnp.stack(cols).reshape(6, _NW, _NCH, _CH)
    idx = xi.transpose(1, 0, 2, 3).reshape(_NW, 6 * _NCH, _CH)
    return _sc_embed(xn16, idx, jnp.asarray(_MAP), Tblob, T01, W4p)


# speedup vs baseline: 1.3329x; 1.3329x over previous
import functools

import numpy as np

import jax
import jax.numpy as jnp
from jax import lax
from jax.experimental import pallas as pl
from jax.experimental.pallas import tpu as pltpu
from jax.experimental.pallas import tpu_sc as plsc

_B = 16384
_NC = 2
_NS = 16
_NW = _NC * _NS
_RPW = _B // _NW
_CH = 128
_NCH = _RPW // _CH
_PIECE = 32

_OUT_D = 322
_STORE_OFF = tuple(16 * g for g in range(20)) + (306,)
_NG = len(_STORE_OFF)

_V56 = 332 + 831


def _build_map() -> np.ndarray:
    spans = (
        (0, 7, 'T01'), (7, 57, 'S0'), (57, 107, 'S1'), (107, 109, 'W4'),
        (109, 159, 'T5'), (159, 209, 'T6'), (209, 259, 'T5'),
        (259, 309, 'T6'), (309, 322, 'XN'),
    )

    def src(c):
        for lo, hi, s in spans:
            if lo <= c < hi:
                return s, c - lo
        raise AssertionError(c)

    vecs = np.zeros((32, 16), np.int64)
    for g in range(7):
        for l in range(16):
            kind, ic = src(_STORE_OFF[g] + l)
            if kind == 'S0':
                vecs[2 * g, l], vecs[2 * g + 1, l] = 0, ic
            elif kind == 'S1':
                vecs[2 * g, l], vecs[2 * g + 1, l] = 128, ic
    for g in range(6, 21):
        for l in range(16):
            kind, ic = src(_STORE_OFF[g] + l)
            if kind in ('T5', 'T6'):
                vecs[14 + (g - 6), l] = ic
    for l in range(7):
        vecs[29, l] = l
    vecs[30, 11], vecs[30, 12] = 0, 1
    for l in range(3, 16):
        vecs[31, l] = l - 3
    tab = np.zeros((4, 128), np.int32)
    for v in range(32):
        tab[v // 8, (v % 8) * 16:(v % 8) * 16 + 16] = vecs[v]
    return tab


_MAP = _build_map()
_CT = tuple(off // 128 for off in _STORE_OFF)
_COFF = tuple(off % 128 for off in _STORE_OFF)


def _body(xn16, idx, vmap, T56, T01, W4p, W2, W3, out,
          idx_v, map_v, t56v, t01v, w4v, stage0, stage1, xnb0, xnb1,
          asm0, asm1, gsem0, gsem1, osem):
    wid = lax.axis_index("s") * _NC + lax.axis_index("c")
    base = wid * _RPW
    stages = (stage0, stage1)
    xnbs = (xnb0, xnb1)
    gsems = (gsem0, gsem1)
    asms = (asm0, asm1)

    pltpu.sync_copy(idx.at[wid], idx_v)
    pltpu.sync_copy(vmap, map_v)
    pltpu.sync_copy(T56, t56v)
    pltpu.sync_copy(T01, t01v)
    pltpu.sync_copy(W4p, w4v)

    fb = [map_v[v // 8, pl.ds((v % 8) * 16, 16)] for v in range(32)]
    lane = jax.lax.iota(jnp.int32, 16)
    m01 = lane < 7
    mw4 = (lane >= 11) & (lane < 13)
    m13 = lane >= 13
    mxn = lane < 3
    m15 = lane < 15
    m1 = lane < 1
    m3 = lane < 3

    def issue_gathers(k):
        stage, gsem = stages[k % 2], gsems[k % 2]
        rows = pl.ds(base + k * _CH, _CH)
        return [
            pltpu.async_copy(W2.at[idx_v.at[1 * _NCH + k]],
                             stage.at[pl.ds(0, _CH)], gsem),
            pltpu.async_copy(W3.at[idx_v.at[2 * _NCH + k]],
                             stage.at[pl.ds(_CH, _CH)], gsem),
            pltpu.async_copy(xn16.at[rows, :], xnbs[k % 2], gsem),
        ]

    def repack_piece(k, piece):
        stage, xnb = stages[k % 2], xnbs[k % 2]
        asm = asms[piece % 2]

        @pl.loop(piece * _PIECE, (piece + 1) * _PIECE)
        def _(r):
            a = r - piece * _PIECE
            rt = a // 8
            s = a - rt * 8
            rvec = jnp.full((16,), r, jnp.int32)

            def bidx(fieldrow):
                return plsc.load_gather(
                    idx_v, [jnp.full((16,), fieldrow, jnp.int32), rvec])

            i01 = bidx(0 * _NCH + k)
            i4 = bidx(3 * _NCH + k)
            i5 = bidx(4 * _NCH + k)
            i6 = bidx(5 * _NCH + k) + 332

            def t56(g, rowvec):
                return plsc.load_gather(t56v, [rowvec, fb[14 + (g - 6)]])

            for g in range(_NG):
                if g < 6:
                    v = plsc.load_gather(stage, [fb[2 * g] + rvec,
                                                 fb[2 * g + 1]])
                    if g == 0:
                        v = jnp.where(
                            m01,
                            plsc.load_gather(t01v, [i01, fb[29]]), v)
                elif g == 6:
                    v = plsc.load_gather(stage, [fb[12] + rvec, fb[13]])
                    v = jnp.where(
                        mw4, plsc.load_gather(w4v, [i4, fb[30]]), v)
                    v = jnp.where(m13, t56(6, i5), v)
                elif g in (7, 8, 14, 15):
                    v = t56(g, i5)
                elif g in (10, 11, 12, 17, 18, 19):
                    v = t56(g, i6)
                elif g == 9:
                    v = t56(g, jnp.where(m15, i5, i6))
                elif g == 13:
                    v = t56(g, jnp.where(m1, i6, i5))
                elif g == 16:
                    v = t56(g, jnp.where(m3, i5, i6))
                else:
                    v = jnp.where(
                        m3, t56(g, i6),
                        plsc.load_gather(xnb, [rvec, fb[31]]))
                asm[rt, _CT[g], s, pl.ds(_COFF[g], 16)] = v

    pend = issue_gathers(0)
    wb = {}
    piece_id = 0
    for k in range(_NCH):
        nxt = issue_gathers(k + 1) if k + 1 < _NCH else []
        for c in pend:
            c.wait()
        pend = nxt
        for piece in range(_CH // _PIECE):
            if piece_id % 2 in wb:
                wb.pop(piece_id % 2).wait()
            repack_piece(k, piece)
            rtb = wid * (_RPW // 8) + k * (_CH // 8) + piece * (_PIECE // 8)
            wb[piece_id % 2] = pltpu.async_copy(
                asms[piece_id % 2],
                out.at[pl.ds(rtb, _PIECE // 8)], osem)
            piece_id += 1
    for c in wb.values():
        c.wait()


_sc_embed = functools.partial(
    pl.kernel,
    out_type=jax.ShapeDtypeStruct((_B // 8, 3, 8, 128), jnp.float32),
    mesh=plsc.VectorSubcoreMesh(core_axis_name="c", subcore_axis_name="s"),
    compiler_params=pltpu.CompilerParams(use_tc_tiling_on_sc=False,
                                         needs_layout_passes=False),
    scratch_types=[
        pltpu.VMEM((6 * _NCH, _CH), jnp.int32),
        pltpu.VMEM((4, 128), jnp.int32),
        pltpu.VMEM((_V56, 50), jnp.float32),
        pltpu.VMEM((40, 16), jnp.float32),
        pltpu.VMEM((4, 16), jnp.float32),
        pltpu.VMEM((2 * _CH, 64), jnp.float32),
        pltpu.VMEM((2 * _CH, 64), jnp.float32),
        pltpu.VMEM((_CH, 16), jnp.float32),
        pltpu.VMEM((_CH, 16), jnp.float32),
        pltpu.VMEM((_PIECE // 8, 3, 8, 128), jnp.float32),
        pltpu.VMEM((_PIECE // 8, 3, 8, 128), jnp.float32),
        pltpu.SemaphoreType.DMA,
        pltpu.SemaphoreType.DMA,
        pltpu.SemaphoreType.DMA,
    ],
)(_body)


def kernel(x_num, x_cat, W0, W1, W2, W3, W4, W5, W6):
    f32 = jnp.float32
    T56 = jnp.concatenate([W5.astype(f32), W6.astype(f32)], axis=0)
    T01 = jnp.concatenate([
        jnp.repeat(W0.astype(f32), 8, axis=0),
        jnp.tile(W1.astype(f32), (5, 1)),
        jnp.zeros((40, 9), f32),
    ], axis=1)
    W4p = jnp.concatenate([W4.astype(f32), jnp.zeros((4, 14), f32)], axis=1)
    xn16 = jnp.concatenate([x_num.astype(f32), jnp.zeros((_B, 3), f32)], axis=1)

    def pad64(Wt):
        return jnp.concatenate(
            [Wt.astype(f32), jnp.zeros((Wt.shape[0], 14), f32)], axis=1)

    W2p, W3p = pad64(W2), pad64(W3)

    xc = x_cat.astype(jnp.int32)
    cols = [xc[:, 0] * 8 + xc[:, 1], xc[:, 2], xc[:, 3], xc[:, 4], xc[:, 5],
            xc[:, 6]]
    xi = jnp.stack(cols).reshape(6, _NW, _NCH, _CH)
    idx = xi.transpose(1, 0, 2, 3).reshape(_NW, 6 * _NCH, _CH)
    tiled = _sc_embed(xn16, idx, jnp.asarray(_MAP), T56, T01, W4p, W2p, W3p)
    return tiled.transpose(0, 2, 1, 3).reshape(_B, 384)[:, :_OUT_D]


# speedup vs baseline: 1.3365x; 1.0027x over previous
import functools

import numpy as np

import jax
import jax.numpy as jnp
from jax import lax
from jax.experimental import pallas as pl
from jax.experimental.pallas import tpu as pltpu
from jax.experimental.pallas import tpu_sc as plsc

_B = 16384
_NC = 2
_NS = 16
_NW = _NC * _NS
_RPW = _B // _NW
_CH = 128
_NCH = _RPW // _CH
_PIECE = 32

_OUT_D = 322
_STORE_OFF = tuple(16 * g for g in range(20)) + (306,)
_NG = len(_STORE_OFF)

_V56 = 332 + 831


def _build_map() -> np.ndarray:
    spans = (
        (0, 7, 'T01'), (7, 57, 'S0'), (57, 107, 'S1'), (107, 109, 'W4'),
        (109, 159, 'T5'), (159, 209, 'T6'), (209, 259, 'T5'),
        (259, 309, 'T6'), (309, 322, 'XN'),
    )

    def src(c):
        for lo, hi, s in spans:
            if lo <= c < hi:
                return s, c - lo
        raise AssertionError(c)

    vecs = np.zeros((32, 16), np.int64)
    for g in range(7):
        for l in range(16):
            kind, ic = src(_STORE_OFF[g] + l)
            if kind == 'S0':
                vecs[2 * g, l], vecs[2 * g + 1, l] = 0, ic
            elif kind == 'S1':
                vecs[2 * g, l], vecs[2 * g + 1, l] = 128, ic
    for g in range(6, 21):
        for l in range(16):
            kind, ic = src(_STORE_OFF[g] + l)
            if kind in ('T5', 'T6'):
                vecs[14 + (g - 6), l] = ic
    for l in range(7):
        vecs[29, l] = l
    vecs[30, 11], vecs[30, 12] = 0, 1
    for l in range(3, 16):
        vecs[31, l] = l - 3
    tab = np.zeros((4, 128), np.int32)
    for v in range(32):
        tab[v // 8, (v % 8) * 16:(v % 8) * 16 + 16] = vecs[v]
    return tab


_MAP = _build_map()
_CT = tuple(off // 128 for off in _STORE_OFF)
_COFF = tuple(off % 128 for off in _STORE_OFF)


def _body(xn2, idx, vmap, T56, Tsm, T23, out,
          idx_v, map_v, t56v, tsmv, stage0, stage1, xnb0, xnb1,
          asm0, asm1, gsem0, gsem1, osem):
    wid = lax.axis_index("s") * _NC + lax.axis_index("c")
    base = wid * _RPW
    stages = (stage0, stage1)
    xnbs = (xnb0, xnb1)
    gsems = (gsem0, gsem1)
    asms = (asm0, asm1)

    for j in range(6):
        pltpu.sync_copy(idx.at[j, wid], idx_v.at[pl.ds(j * _NCH, _NCH)])
    pltpu.sync_copy(vmap, map_v)
    pltpu.sync_copy(T56, t56v)
    pltpu.sync_copy(Tsm, tsmv)

    fb = [map_v[v // 8, pl.ds((v % 8) * 16, 16)] for v in range(32)]
    lane = jax.lax.iota(jnp.int32, 16)
    m01 = lane < 7
    mw4 = (lane >= 11) & (lane < 13)
    m13 = lane >= 13
    mxn = lane < 3
    m15 = lane < 15
    m1 = lane < 1
    m3 = lane < 3

    def issue_gathers(k):
        stage, gsem = stages[k % 2], gsems[k % 2]
        rows2 = pl.ds((base + k * _CH) // 8, _CH // 8)
        return [
            pltpu.async_copy(T23.at[idx_v.at[1 * _NCH + k]],
                             stage.at[pl.ds(0, _CH)], gsem),
            pltpu.async_copy(T23.at[idx_v.at[2 * _NCH + k]],
                             stage.at[pl.ds(_CH, _CH)], gsem),
            pltpu.async_copy(xn2.at[rows2, :], xnbs[k % 2], gsem),
        ]

    def repack_piece(k, piece):
        stage, xnb = stages[k % 2], xnbs[k % 2]
        asm = asms[piece % 2]

        @pl.loop(piece * _PIECE, (piece + 1) * _PIECE)
        def _(r):
            a = r - piece * _PIECE
            rt = a // 8
            s = a - rt * 8
            rchunk_t = r // 8
            ssub = r - rchunk_t * 8
            rvec = jnp.full((16,), r, jnp.int32)

            def bidx(fieldrow):
                return plsc.load_gather(
                    idx_v, [jnp.full((16,), fieldrow, jnp.int32), rvec])

            i01 = bidx(0 * _NCH + k)
            i4 = bidx(3 * _NCH + k)
            i5 = bidx(4 * _NCH + k)
            i6 = bidx(5 * _NCH + k) + 332

            def t56(g, rowvec):
                return plsc.load_gather(t56v, [rowvec, fb[14 + (g - 6)]])

            for g in range(_NG):
                if g < 6:
                    v = plsc.load_gather(stage, [fb[2 * g] + rvec,
                                                 fb[2 * g + 1]])
                    if g == 0:
                        v = jnp.where(
                            m01,
                            plsc.load_gather(tsmv, [i01, fb[29]]), v)
                elif g == 6:
                    v = plsc.load_gather(stage, [fb[12] + rvec, fb[13]])
                    v = jnp.where(
                        mw4, plsc.load_gather(tsmv, [i4, fb[30]]), v)
                    v = jnp.where(m13, t56(6, i5), v)
                elif g in (7, 8, 14, 15):
                    v = t56(g, i5)
                elif g in (10, 11, 12, 17, 18, 19):
                    v = t56(g, i6)
                elif g == 9:
                    v = t56(g, jnp.where(m15, i5, i6))
                elif g == 13:
                    v = t56(g, jnp.where(m1, i6, i5))
                elif g == 16:
                    v = t56(g, jnp.where(m3, i5, i6))
                else:
                    v = jnp.where(
                        m3, t56(g, i6),
                        plsc.load_gather(
                            xnb,
                            [jnp.full((16,), rchunk_t, jnp.int32),
                             fb[31] + jnp.full((16,), ssub * 16,
                                               jnp.int32)]))
                asm[rt, _CT[g], s, pl.ds(_COFF[g], 16)] = v

    pend = issue_gathers(0)
    wb = {}
    piece_id = 0
    for k in range(_NCH):
        nxt = issue_gathers(k + 1) if k + 1 < _NCH else []
        for c in pend:
            c.wait()
        pend = nxt
        for piece in range(_CH // _PIECE):
            if piece_id % 2 in wb:
                wb.pop(piece_id % 2).wait()
            repack_piece(k, piece)
            rtb = wid * (_RPW // 8) + k * (_CH // 8) + piece * (_PIECE // 8)
            wb[piece_id % 2] = pltpu.async_copy(
                asms[piece_id % 2],
                out.at[pl.ds(rtb, _PIECE // 8)], osem)
            piece_id += 1
    for c in wb.values():
        c.wait()


_sc_embed = functools.partial(
    pl.kernel,
    out_type=jax.ShapeDtypeStruct((_B // 8, 3, 8, 128), jnp.float32),
    mesh=plsc.VectorSubcoreMesh(core_axis_name="c", subcore_axis_name="s"),
    compiler_params=pltpu.CompilerParams(use_tc_tiling_on_sc=False,
                                         needs_layout_passes=False),
    scratch_types=[
        pltpu.VMEM((6 * _NCH, _CH), jnp.int32),
        pltpu.VMEM((4, 128), jnp.int32),
        pltpu.VMEM((_V56, 50), jnp.float32),
        pltpu.VMEM((44, 16), jnp.float32),
        pltpu.VMEM((2 * _CH, 64), jnp.float32),
        pltpu.VMEM((2 * _CH, 64), jnp.float32),
        pltpu.VMEM((_CH // 8, 128), jnp.float32),
        pltpu.VMEM((_CH // 8, 128), jnp.float32),
        pltpu.VMEM((_PIECE // 8, 3, 8, 128), jnp.float32),
        pltpu.VMEM((_PIECE // 8, 3, 8, 128), jnp.float32),
        pltpu.SemaphoreType.DMA,
        pltpu.SemaphoreType.DMA,
        pltpu.SemaphoreType.DMA,
    ],
)(_body)


def kernel(x_num, x_cat, W0, W1, W2, W3, W4, W5, W6):
    f32 = jnp.float32
    T56 = jnp.concatenate([W5.astype(f32), W6.astype(f32)], axis=0)
    Tsm = jnp.concatenate([
        jnp.concatenate([
            jnp.repeat(W0.astype(f32), 8, axis=0),
            jnp.tile(W1.astype(f32), (5, 1)),
            jnp.zeros((40, 9), f32),
        ], axis=1),
        jnp.concatenate([W4.astype(f32), jnp.zeros((4, 14), f32)], axis=1),
    ], axis=0)
    T23 = jnp.concatenate([
        jnp.concatenate([W2.astype(f32), W3.astype(f32)], axis=0),
        jnp.zeros((1906, 14), f32),
    ], axis=1)
    xn2 = jnp.concatenate(
        [x_num.astype(f32), jnp.zeros((_B, 3), f32)], axis=1
    ).reshape(_B // 8, 128)

    xc = x_cat.astype(jnp.int32)
    cols = [xc[:, 0] * 8 + xc[:, 1], xc[:, 2], xc[:, 3] + 1218,
            xc[:, 4] + 40, xc[:, 5], xc[:, 6]]
    idx = jnp.stack(cols).reshape(6, _NW, _NCH, _CH)
    tiled = _sc_embed(xn2, idx, jnp.asarray(_MAP), T56, Tsm, T23)
    return tiled.transpose(0, 2, 1, 3).reshape(_B, 384)[:, :_OUT_D]
